# baseline (speedup 1.0000x reference)
import functools

import jax
import jax.numpy as jnp
from jax import lax
from jax.experimental import pallas as pl
from jax.experimental.pallas import tpu as pltpu
from jax.experimental.pallas import tpu_sc as plsc

_N = 10000
_NPAD = 10240
_E = 320000
_F = 128
_H = 128
_OUT = 64
_NC = 2
_NS = 16
_CH = 128
_WPT = 80
_WPH = 40
_EROWS = _NC * _NS * _WPT
_EPAD = _EROWS * _CH
_ROWS_PT = _NPAD // _NS
_MBLK = 128


def _seg_body(p, src, dst, zeros, out,
              srcs, dsts, rows0, rows1, acc, sg0, sg1, ss0, ss1):
    c = lax.axis_index("c")
    s = lax.axis_index("s")
    t = c * _NS + s

    pltpu.sync_copy(zeros, acc.at[pl.ds(s * _ROWS_PT, _ROWS_PT)])
    plsc.subcore_barrier()

    for ph in range(2):
        row0 = t * _WPT + ph * _WPH
        pltpu.sync_copy(src.at[pl.ds(row0, _WPH)], srcs)
        pltpu.sync_copy(dst.at[pl.ds(row0, _WPH)], dsts)

        pltpu.async_copy(p.at[srcs.at[0]], rows0, sg0).wait()
        pltpu.async_copy(p.at[srcs.at[1]], rows1, sg1)
        pltpu.async_copy(rows0, acc.at[dsts.at[0]], ss0, add=True)

        def _pair(i, _):
            g = 2 * i + 1
            pltpu.make_async_copy(p.at[srcs.at[g]], rows1, sg1).wait()
            pltpu.make_async_copy(rows0, acc.at[dsts.at[g - 1]], ss0).wait()
            pltpu.async_copy(p.at[srcs.at[g + 1]], rows0, sg0)
            pltpu.async_copy(rows1, acc.at[dsts.at[g]], ss1, add=True)
            pltpu.make_async_copy(p.at[srcs.at[g + 1]], rows0, sg0).wait()
            pltpu.make_async_copy(rows1, acc.at[dsts.at[g]], ss1).wait()
            pltpu.async_copy(p.at[srcs.at[g + 2]], rows1, sg1)
            pltpu.async_copy(rows0, acc.at[dsts.at[g + 1]], ss0, add=True)
            return 0

        lax.fori_loop(0, (_WPH - 2) // 2, _pair, 0)

        g = _WPH - 1
        pltpu.make_async_copy(p.at[srcs.at[g]], rows1, sg1).wait()
        pltpu.make_async_copy(rows0, acc.at[dsts.at[g - 1]], ss0).wait()
        pltpu.async_copy(rows1, acc.at[dsts.at[g]], ss1, add=True)
        pltpu.make_async_copy(rows1, acc.at[dsts.at[g]], ss1).wait()

    plsc.subcore_barrier()
    pltpu.sync_copy(acc.at[pl.ds(s * _ROWS_PT, _ROWS_PT)],
                    out.at[c, pl.ds(s * _ROWS_PT, _ROWS_PT)])


def _cnt_body(dst, ones, zeros, out, dsts, ones_v, acc, ss):
    c = lax.axis_index("c")
    s = lax.axis_index("s")
    t = c * _NS + s

    pltpu.sync_copy(ones, ones_v)
    pltpu.sync_copy(zeros, acc.at[pl.ds(s * _ROWS_PT, _ROWS_PT)])
    pltpu.sync_copy(dst.at[pl.ds(t * _WPT, _WPT)], dsts)
    plsc.subcore_barrier()

    def _grp(i, _):
        for j in range(8):
            pltpu.async_copy(ones_v, acc.at[dsts.at[8 * i + j]], ss,
                             add=True)
        for j in range(8):
            pltpu.make_async_copy(ones_v, acc.at[dsts.at[8 * i + j]],
                                  ss).wait()
        return 0

    lax.fori_loop(0, _WPT // 8, _grp, 0)
    plsc.subcore_barrier()
    pltpu.sync_copy(acc.at[pl.ds(s * _ROWS_PT, _ROWS_PT)],
                    out.at[c, pl.ds(s * _ROWS_PT, _ROWS_PT)])


@functools.cache
def _seg_call():
    mesh = plsc.VectorSubcoreMesh(core_axis_name="c", subcore_axis_name="s",
                                  num_cores=_NC, num_subcores=_NS)
    return pl.kernel(
        _seg_body,
        out_type=jax.ShapeDtypeStruct((_NC, _NPAD, _H), jnp.float32),
        mesh=mesh,
        scratch_types=[
            pltpu.VMEM((_WPH, _CH), jnp.int32),
            pltpu.VMEM((_WPH, _CH), jnp.int32),
            pltpu.VMEM((_CH, _H), jnp.float32),
            pltpu.VMEM((_CH, _H), jnp.float32),
            pltpu.VMEM_SHARED((_NPAD, _H), jnp.float32),
            pltpu.SemaphoreType.DMA,
            pltpu.SemaphoreType.DMA,
            pltpu.SemaphoreType.DMA,
            pltpu.SemaphoreType.DMA,
        ],
    )


@functools.cache
def _cnt_call():
    mesh = plsc.VectorSubcoreMesh(core_axis_name="c", subcore_axis_name="s",
                                  num_cores=_NC, num_subcores=_NS)
    return pl.kernel(
        _cnt_body,
        out_type=jax.ShapeDtypeStruct((_NC, _NPAD, _H), jnp.float32),
        mesh=mesh,
        scratch_types=[
            pltpu.VMEM((_WPT, _CH), jnp.int32),
            pltpu.VMEM((_CH, _H), jnp.float32),
            pltpu.VMEM_SHARED((_NPAD, _H), jnp.float32),
            pltpu.SemaphoreType.DMA,
        ],
    )


def _mm_p_kernel(h_ref, w_ref, o_ref):
    o_ref[...] = jnp.dot(h_ref[...], w_ref[...],
                         preferred_element_type=jnp.float32)


def _mm_p(h, wl):
    return pl.pallas_call(
        _mm_p_kernel,
        grid=(_NPAD // _MBLK,),
        in_specs=[
            pl.BlockSpec((_MBLK, _H), lambda i: (i, 0)),
            pl.BlockSpec((_H, _H), lambda i: (0, 0)),
        ],
        out_specs=pl.BlockSpec((_MBLK, _H), lambda i: (i, 0)),
        out_shape=jax.ShapeDtypeStruct((_NPAD, _H), jnp.float32),
    )(h, wl)


def _combine_kernel(s0, s1, c0, c1, h, wr, bl, o_ref):
    cnt = c0[:, 0:1] + c1[:, 0:1]
    inv = 1.0 / jnp.maximum(cnt, 1.0)
    mean = (s0[...] + s1[...]) * inv
    mm = jnp.dot(h[...], wr[...], preferred_element_type=jnp.float32)
    o_ref[...] = jnp.maximum(mean + mm + bl[...], 0.0)


def _combine(s0, s1, c0, c1, h, wr, bl):
    return pl.pallas_call(
        _combine_kernel,
        grid=(_NPAD // _MBLK,),
        in_specs=[
            pl.BlockSpec((_MBLK, _H), lambda i: (i, 0)),
            pl.BlockSpec((_MBLK, _H), lambda i: (i, 0)),
            pl.BlockSpec((_MBLK, _H), lambda i: (i, 0)),
            pl.BlockSpec((_MBLK, _H), lambda i: (i, 0)),
            pl.BlockSpec((_MBLK, _H), lambda i: (i, 0)),
            pl.BlockSpec((_H, _H), lambda i: (0, 0)),
            pl.BlockSpec((1, _H), lambda i: (0, 0)),
        ],
        out_specs=pl.BlockSpec((_MBLK, _H), lambda i: (i, 0)),
        out_shape=jax.ShapeDtypeStruct((_NPAD, _H), jnp.float32),
    )(s0, s1, c0, c1, h, wr, bl)


def _fc_kernel(h1, h2, h3, w1, w2, w3, b, o_ref):
    acc = jnp.dot(h1[...], w1[...], preferred_element_type=jnp.float32)
    acc += jnp.dot(h2[...], w2[...], preferred_element_type=jnp.float32)
    acc += jnp.dot(h3[...], w3[...], preferred_element_type=jnp.float32)
    o_ref[...] = acc + b[...]


def _fc(h1, h2, h3, w1, w2, w3, b):
    return pl.pallas_call(
        _fc_kernel,
        grid=(_NPAD // _MBLK,),
        in_specs=[
            pl.BlockSpec((_MBLK, _H), lambda i: (i, 0)),
            pl.BlockSpec((_MBLK, _H), lambda i: (i, 0)),
            pl.BlockSpec((_MBLK, _H), lambda i: (i, 0)),
            pl.BlockSpec((_H, _OUT), lambda i: (0, 0)),
            pl.BlockSpec((_H, _OUT), lambda i: (0, 0)),
            pl.BlockSpec((_H, _OUT), lambda i: (0, 0)),
            pl.BlockSpec((1, _OUT), lambda i: (0, 0)),
        ],
        out_specs=pl.BlockSpec((_MBLK, _OUT), lambda i: (i, 0)),
        out_shape=jax.ShapeDtypeStruct((_NPAD, _OUT), jnp.float32),
    )(h1, h2, h3, w1, w2, w3, b)


def kernel(x, edge_index, Wl0, bl0, Wr0, Wl1, bl1, Wr1, Wl2, bl2, Wr2,
           W_fc, b_fc):
    fill = jnp.full((_EPAD - _E,), _NPAD - 1, jnp.int32)
    src2d = jnp.concatenate([edge_index[0], fill]).reshape(_EROWS, _CH)
    dst2d = jnp.concatenate([edge_index[1], fill]).reshape(_EROWS, _CH)

    xpad = jnp.pad(x, ((0, _NPAD - _N), (0, 0)))
    zeros_h = jnp.zeros((_ROWS_PT, _H), jnp.float32)
    ones_h = jnp.ones((_CH, _H), jnp.float32)

    cnt = _cnt_call()(dst2d, ones_h, zeros_h)

    h = xpad
    hs = []
    for Wl, bl, Wr in ((Wl0, bl0, Wr0), (Wl1, bl1, Wr1), (Wl2, bl2, Wr2)):
        p = _mm_p(h, Wl)
        ssum = _seg_call()(p, src2d, dst2d, zeros_h)
        h = _combine(ssum[0], ssum[1], cnt[0], cnt[1], h, Wr,
                     bl.reshape(1, _H))
        hs.append(h)

    out = _fc(hs[0], hs[1], hs[2], W_fc[0:_H], W_fc[_H:2 * _H],
              W_fc[2 * _H:3 * _H], b_fc.reshape(1, _OUT))
    return out[:_N]


# speedup vs baseline: 1.9991x; 1.9991x over previous
import functools

import jax
import jax.numpy as jnp
from jax import lax
from jax.experimental import pallas as pl
from jax.experimental.pallas import tpu as pltpu
from jax.experimental.pallas import tpu_sc as plsc

_N = 10000
_NPAD = 10240
_E = 320000
_F = 128
_H = 128
_OUT = 64
_NC = 2
_NS = 16
_CH = 128
_WPT = 80
_WPH = 40
_EROWS = _NC * _NS * _WPT
_EPAD = _EROWS * _CH
_ROWS_PT = _NPAD // _NS
_MBLK = 128


def _seg_body(p, src, dst, zeros, out,
              srcs, dsts, rows0, rows1, acc, sg0, sg1, ss0, ss1):
    c = lax.axis_index("c")
    s = lax.axis_index("s")
    t = c * _NS + s

    pltpu.sync_copy(zeros, acc.at[pl.ds(s * _ROWS_PT, _ROWS_PT)])
    plsc.subcore_barrier()

    for ph in range(2):
        row0 = t * _WPT + ph * _WPH
        pltpu.sync_copy(src.at[pl.ds(row0, _WPH)], srcs)
        pltpu.sync_copy(dst.at[pl.ds(row0, _WPH)], dsts)

        pltpu.async_copy(p.at[srcs.at[0]], rows0, sg0).wait()
        pltpu.async_copy(p.at[srcs.at[1]], rows1, sg1)
        pltpu.async_copy(rows0, acc.at[dsts.at[0]], ss0, add=True)

        def _pair(i, _):
            g = 2 * i + 1
            pltpu.make_async_copy(p.at[srcs.at[g]], rows1, sg1).wait()
            pltpu.make_async_copy(rows0, acc.at[dsts.at[g - 1]], ss0).wait()
            pltpu.async_copy(p.at[srcs.at[g + 1]], rows0, sg0)
            pltpu.async_copy(rows1, acc.at[dsts.at[g]], ss1, add=True)
            pltpu.make_async_copy(p.at[srcs.at[g + 1]], rows0, sg0).wait()
            pltpu.make_async_copy(rows1, acc.at[dsts.at[g]], ss1).wait()
            pltpu.async_copy(p.at[srcs.at[g + 2]], rows1, sg1)
            pltpu.async_copy(rows0, acc.at[dsts.at[g + 1]], ss0, add=True)
            return 0

        lax.fori_loop(0, (_WPH - 2) // 2, _pair, 0)

        g = _WPH - 1
        pltpu.make_async_copy(p.at[srcs.at[g]], rows1, sg1).wait()
        pltpu.make_async_copy(rows0, acc.at[dsts.at[g - 1]], ss0).wait()
        pltpu.async_copy(rows1, acc.at[dsts.at[g]], ss1, add=True)
        pltpu.make_async_copy(rows1, acc.at[dsts.at[g]], ss1).wait()

    plsc.subcore_barrier()
    pltpu.sync_copy(acc.at[pl.ds(s * _ROWS_PT, _ROWS_PT)],
                    out.at[c, pl.ds(s * _ROWS_PT, _ROWS_PT)])


def _cnt_body(dst, ones, zeros, out, dsts, ones_v, acc, ss):
    c = lax.axis_index("c")
    s = lax.axis_index("s")
    t = c * _NS + s

    pltpu.sync_copy(ones, ones_v)
    pltpu.sync_copy(zeros, acc.at[pl.ds(s * _ROWS_PT, _ROWS_PT)])
    pltpu.sync_copy(dst.at[pl.ds(t * _WPT, _WPT)], dsts)
    plsc.subcore_barrier()

    def _grp(i, _):
        for j in range(8):
            pltpu.async_copy(ones_v, acc.at[dsts.at[8 * i + j]], ss,
                             add=True)
        for j in range(8):
            pltpu.make_async_copy(ones_v, acc.at[dsts.at[8 * i + j]],
                                  ss).wait()
        return 0

    lax.fori_loop(0, _WPT // 8, _grp, 0)
    plsc.subcore_barrier()
    pltpu.sync_copy(acc.at[pl.ds(s * _ROWS_PT, _ROWS_PT)],
                    out.at[c, pl.ds(s * _ROWS_PT, _ROWS_PT)])


@functools.cache
def _seg_call():
    mesh = plsc.VectorSubcoreMesh(core_axis_name="c", subcore_axis_name="s",
                                  num_cores=_NC, num_subcores=_NS)
    return pl.kernel(
        _seg_body,
        out_type=jax.ShapeDtypeStruct((_NC, _NPAD, _H), jnp.float32),
        mesh=mesh,
        scratch_types=[
            pltpu.VMEM((_WPH, _CH), jnp.int32),
            pltpu.VMEM((_WPH, _CH), jnp.int32),
            pltpu.VMEM((_CH, _H), jnp.float32),
            pltpu.VMEM((_CH, _H), jnp.float32),
            pltpu.VMEM_SHARED((_NPAD, _H), jnp.float32),
            pltpu.SemaphoreType.DMA,
            pltpu.SemaphoreType.DMA,
            pltpu.SemaphoreType.DMA,
            pltpu.SemaphoreType.DMA,
        ],
    )


@functools.cache
def _cnt_call():
    mesh = plsc.VectorSubcoreMesh(core_axis_name="c", subcore_axis_name="s",
                                  num_cores=_NC, num_subcores=_NS)
    return pl.kernel(
        _cnt_body,
        out_type=jax.ShapeDtypeStruct((_NC, _NPAD, _H), jnp.float32),
        mesh=mesh,
        scratch_types=[
            pltpu.VMEM((_WPT, _CH), jnp.int32),
            pltpu.VMEM((_CH, _H), jnp.float32),
            pltpu.VMEM_SHARED((_NPAD, _H), jnp.float32),
            pltpu.SemaphoreType.DMA,
        ],
    )


def _mm_p_kernel(h_ref, w_ref, o_ref):
    o_ref[...] = jnp.dot(h_ref[...], w_ref[...],
                         preferred_element_type=jnp.float32)


def _mm_p(h, wl):
    return pl.pallas_call(
        _mm_p_kernel,
        grid=(_NPAD // _MBLK,),
        in_specs=[
            pl.BlockSpec((_MBLK, _H), lambda i: (i, 0)),
            pl.BlockSpec((_H, _H), lambda i: (0, 0)),
        ],
        out_specs=pl.BlockSpec((_MBLK, _H), lambda i: (i, 0)),
        out_shape=jax.ShapeDtypeStruct((_NPAD, _H), jnp.float32),
    )(h, wl)


def _combine_kernel(s0, s1, c0, c1, h, wr, bl, o_ref):
    cnt = c0[:, 0:1] + c1[:, 0:1]
    inv = 1.0 / jnp.maximum(cnt, 1.0)
    mean = (s0[...] + s1[...]) * inv
    mm = jnp.dot(h[...], wr[...], preferred_element_type=jnp.float32)
    o_ref[...] = jnp.maximum(mean + mm + bl[...], 0.0)


def _combine(s0, s1, c0, c1, h, wr, bl):
    return pl.pallas_call(
        _combine_kernel,
        grid=(_NPAD // _MBLK,),
        in_specs=[
            pl.BlockSpec((_MBLK, _H), lambda i: (i, 0)),
            pl.BlockSpec((_MBLK, _H), lambda i: (i, 0)),
            pl.BlockSpec((_MBLK, _H), lambda i: (i, 0)),
            pl.BlockSpec((_MBLK, _H), lambda i: (i, 0)),
            pl.BlockSpec((_MBLK, _H), lambda i: (i, 0)),
            pl.BlockSpec((_H, _H), lambda i: (0, 0)),
            pl.BlockSpec((1, _H), lambda i: (0, 0)),
        ],
        out_specs=pl.BlockSpec((_MBLK, _H), lambda i: (i, 0)),
        out_shape=jax.ShapeDtypeStruct((_NPAD, _H), jnp.float32),
    )(s0, s1, c0, c1, h, wr, bl)


def _fc_kernel(h1, h2, h3, w1, w2, w3, b, o_ref):
    acc = jnp.dot(h1[...], w1[...], preferred_element_type=jnp.float32)
    acc += jnp.dot(h2[...], w2[...], preferred_element_type=jnp.float32)
    acc += jnp.dot(h3[...], w3[...], preferred_element_type=jnp.float32)
    o_ref[...] = acc + b[...]


def _fc(h1, h2, h3, w1, w2, w3, b):
    return pl.pallas_call(
        _fc_kernel,
        grid=(_NPAD // _MBLK,),
        in_specs=[
            pl.BlockSpec((_MBLK, _H), lambda i: (i, 0)),
            pl.BlockSpec((_MBLK, _H), lambda i: (i, 0)),
            pl.BlockSpec((_MBLK, _H), lambda i: (i, 0)),
            pl.BlockSpec((_H, _OUT), lambda i: (0, 0)),
            pl.BlockSpec((_H, _OUT), lambda i: (0, 0)),
            pl.BlockSpec((_H, _OUT), lambda i: (0, 0)),
            pl.BlockSpec((1, _OUT), lambda i: (0, 0)),
        ],
        out_specs=pl.BlockSpec((_MBLK, _OUT), lambda i: (i, 0)),
        out_shape=jax.ShapeDtypeStruct((_NPAD, _OUT), jnp.float32),
    )(h1, h2, h3, w1, w2, w3, b)


def kernel(x, edge_index, Wl0, bl0, Wr0, Wl1, bl1, Wr1, Wl2, bl2, Wr2,
           W_fc, b_fc):
    fill = _N + jnp.arange(_EPAD - _E, dtype=jnp.int32) % (_NPAD - _N)
    src2d = jnp.concatenate([edge_index[0], fill]).reshape(_EROWS, _CH)
    dst2d = jnp.concatenate([edge_index[1], fill]).reshape(_EROWS, _CH)

    xpad = jnp.pad(x, ((0, _NPAD - _N), (0, 0)))
    zeros_h = jnp.zeros((_ROWS_PT, _H), jnp.float32)
    ones_h = jnp.ones((_CH, _H), jnp.float32)

    cnt = _cnt_call()(dst2d, ones_h, zeros_h)

    h = xpad
    hs = []
    for Wl, bl, Wr in ((Wl0, bl0, Wr0), (Wl1, bl1, Wr1), (Wl2, bl2, Wr2)):
        p = _mm_p(h, Wl)
        ssum = _seg_call()(p, src2d, dst2d, zeros_h)
        h = _combine(ssum[0], ssum[1], cnt[0], cnt[1], h, Wr,
                     bl.reshape(1, _H))
        hs.append(h)

    out = _fc(hs[0], hs[1], hs[2], W_fc[0:_H], W_fc[_H:2 * _H],
              W_fc[2 * _H:3 * _H], b_fc.reshape(1, _OUT))
    return out[:_N]


# speedup vs baseline: 2.3203x; 1.1607x over previous
import functools

import jax
import jax.numpy as jnp
from jax import lax
from jax.experimental import pallas as pl
from jax.experimental.pallas import tpu as pltpu
from jax.experimental.pallas import tpu_sc as plsc

_N = 10000
_NPAD = 10240
_E = 320000
_F = 128
_H = 128
_OUT = 64
_NC = 2
_NS = 16
_CH = 128
_WPT = 80
_WPH = 40
_EROWS = _NC * _NS * _WPT
_EPAD = _EROWS * _CH
_ROWS_PT = _NPAD // _NS
_MBLK = 128


def _seg_body(p, src, dst, zeros, out,
              srcs, dsts, rows0, rows1, acc, sg0, sg1, ss0, ss1):
    c = lax.axis_index("c")
    s = lax.axis_index("s")
    t = c * _NS + s

    pltpu.sync_copy(zeros, acc.at[pl.ds(s * _ROWS_PT, _ROWS_PT)])
    plsc.subcore_barrier()

    for ph in range(2):
        row0 = t * _WPT + ph * _WPH
        pltpu.sync_copy(src.at[pl.ds(row0, _WPH)], srcs)
        pltpu.sync_copy(dst.at[pl.ds(row0, _WPH)], dsts)

        pltpu.async_copy(p.at[srcs.at[0]], rows0, sg0).wait()
        pltpu.async_copy(p.at[srcs.at[1]], rows1, sg1)
        pltpu.async_copy(rows0, acc.at[dsts.at[0]], ss0, add=True)

        def _pair(i, _):
            g = 2 * i + 1
            pltpu.make_async_copy(p.at[srcs.at[g]], rows1, sg1).wait()
            pltpu.make_async_copy(rows0, acc.at[dsts.at[g - 1]], ss0).wait()
            pltpu.async_copy(p.at[srcs.at[g + 1]], rows0, sg0)
            pltpu.async_copy(rows1, acc.at[dsts.at[g]], ss1, add=True)
            pltpu.make_async_copy(p.at[srcs.at[g + 1]], rows0, sg0).wait()
            pltpu.make_async_copy(rows1, acc.at[dsts.at[g]], ss1).wait()
            pltpu.async_copy(p.at[srcs.at[g + 2]], rows1, sg1)
            pltpu.async_copy(rows0, acc.at[dsts.at[g + 1]], ss0, add=True)
            return 0

        lax.fori_loop(0, (_WPH - 2) // 2, _pair, 0)

        g = _WPH - 1
        pltpu.make_async_copy(p.at[srcs.at[g]], rows1, sg1).wait()
        pltpu.make_async_copy(rows0, acc.at[dsts.at[g - 1]], ss0).wait()
        pltpu.async_copy(rows1, acc.at[dsts.at[g]], ss1, add=True)
        pltpu.make_async_copy(rows1, acc.at[dsts.at[g]], ss1).wait()

    plsc.subcore_barrier()
    pltpu.sync_copy(acc.at[pl.ds(s * _ROWS_PT, _ROWS_PT)],
                    out.at[c, pl.ds(s * _ROWS_PT, _ROWS_PT)])


def _cnt_body(dst, ones, zeros, out, dsts, ones_v, acc, ss):
    c = lax.axis_index("c")
    s = lax.axis_index("s")
    t = c * _NS + s

    pltpu.sync_copy(ones, ones_v)
    pltpu.sync_copy(zeros, acc.at[pl.ds(s * _ROWS_PT, _ROWS_PT)])
    pltpu.sync_copy(dst.at[pl.ds(t * _WPT, _WPT)], dsts)
    plsc.subcore_barrier()

    def _grp(i, _):
        for j in range(8):
            pltpu.async_copy(ones_v, acc.at[dsts.at[8 * i + j]], ss,
                             add=True)
        for j in range(8):
            pltpu.make_async_copy(ones_v, acc.at[dsts.at[8 * i + j]],
                                  ss).wait()
        return 0

    lax.fori_loop(0, _WPT // 8, _grp, 0)
    plsc.subcore_barrier()
    pltpu.sync_copy(acc.at[pl.ds(s * _ROWS_PT, _ROWS_PT)],
                    out.at[c, pl.ds(s * _ROWS_PT, _ROWS_PT)])


@functools.cache
def _seg_call():
    mesh = plsc.VectorSubcoreMesh(core_axis_name="c", subcore_axis_name="s",
                                  num_cores=_NC, num_subcores=_NS)
    return pl.kernel(
        _seg_body,
        out_type=jax.ShapeDtypeStruct((_NC, _NPAD, _H), jnp.float32),
        mesh=mesh,
        scratch_types=[
            pltpu.VMEM((_WPH, _CH), jnp.int32),
            pltpu.VMEM((_WPH, _CH), jnp.int32),
            pltpu.VMEM((_CH, _H), jnp.float32),
            pltpu.VMEM((_CH, _H), jnp.float32),
            pltpu.VMEM_SHARED((_NPAD, _H), jnp.float32),
            pltpu.SemaphoreType.DMA,
            pltpu.SemaphoreType.DMA,
            pltpu.SemaphoreType.DMA,
            pltpu.SemaphoreType.DMA,
        ],
    )


@functools.cache
def _cnt_call():
    mesh = plsc.VectorSubcoreMesh(core_axis_name="c", subcore_axis_name="s",
                                  num_cores=_NC, num_subcores=_NS)
    return pl.kernel(
        _cnt_body,
        out_type=jax.ShapeDtypeStruct((_NC, _NPAD, _H), jnp.float32),
        mesh=mesh,
        scratch_types=[
            pltpu.VMEM((_WPT, _CH), jnp.int32),
            pltpu.VMEM((_CH, _H), jnp.float32),
            pltpu.VMEM_SHARED((_NPAD, _H), jnp.float32),
            pltpu.SemaphoreType.DMA,
        ],
    )


def _mm_p_kernel(h_ref, w_ref, o_ref):
    o_ref[...] = jnp.dot(h_ref[...], w_ref[...],
                         preferred_element_type=jnp.float32)


def _mm_p(h, wl):
    return pl.pallas_call(
        _mm_p_kernel,
        grid=(_NPAD // _MBLK,),
        in_specs=[
            pl.BlockSpec((_MBLK, _H), lambda i: (i, 0)),
            pl.BlockSpec((_H, _H), lambda i: (0, 0)),
        ],
        out_specs=pl.BlockSpec((_MBLK, _H), lambda i: (i, 0)),
        out_shape=jax.ShapeDtypeStruct((_NPAD, _H), jnp.float32),
    )(h, wl)


def _combine_proj_kernel(s0, s1, c0, c1, h, wr, bl, wl_n, h_ref, p_ref):
    cnt = c0[:, 0:1] + c1[:, 0:1]
    inv = 1.0 / jnp.maximum(cnt, 1.0)
    mean = (s0[...] + s1[...]) * inv
    mm = jnp.dot(h[...], wr[...], preferred_element_type=jnp.float32)
    hn = jnp.maximum(mean + mm + bl[...], 0.0)
    h_ref[...] = hn
    p_ref[...] = jnp.dot(hn, wl_n[...], preferred_element_type=jnp.float32)


def _combine_proj(s0, s1, c0, c1, h, wr, bl, wl_n):
    return pl.pallas_call(
        _combine_proj_kernel,
        grid=(_NPAD // _MBLK,),
        in_specs=[
            pl.BlockSpec((_MBLK, _H), lambda i: (i, 0)),
            pl.BlockSpec((_MBLK, _H), lambda i: (i, 0)),
            pl.BlockSpec((_MBLK, _H), lambda i: (i, 0)),
            pl.BlockSpec((_MBLK, _H), lambda i: (i, 0)),
            pl.BlockSpec((_MBLK, _H), lambda i: (i, 0)),
            pl.BlockSpec((_H, _H), lambda i: (0, 0)),
            pl.BlockSpec((1, _H), lambda i: (0, 0)),
            pl.BlockSpec((_H, _H), lambda i: (0, 0)),
        ],
        out_specs=[
            pl.BlockSpec((_MBLK, _H), lambda i: (i, 0)),
            pl.BlockSpec((_MBLK, _H), lambda i: (i, 0)),
        ],
        out_shape=[
            jax.ShapeDtypeStruct((_NPAD, _H), jnp.float32),
            jax.ShapeDtypeStruct((_NPAD, _H), jnp.float32),
        ],
    )(s0, s1, c0, c1, h, wr, bl, wl_n)


def _combine_fc_kernel(s0, s1, c0, c1, h, wr, bl, h1, h2,
                       w1, w2, w3, b, o_ref):
    cnt = c0[:, 0:1] + c1[:, 0:1]
    inv = 1.0 / jnp.maximum(cnt, 1.0)
    mean = (s0[...] + s1[...]) * inv
    mm = jnp.dot(h[...], wr[...], preferred_element_type=jnp.float32)
    h3 = jnp.maximum(mean + mm + bl[...], 0.0)
    acc = jnp.dot(h1[...], w1[...], preferred_element_type=jnp.float32)
    acc += jnp.dot(h2[...], w2[...], preferred_element_type=jnp.float32)
    acc += jnp.dot(h3, w3[...], preferred_element_type=jnp.float32)
    o_ref[...] = acc + b[...]


def _combine_fc(s0, s1, c0, c1, h, wr, bl, h1, h2, w1, w2, w3, b):
    return pl.pallas_call(
        _combine_fc_kernel,
        grid=(_NPAD // _MBLK,),
        in_specs=[
            pl.BlockSpec((_MBLK, _H), lambda i: (i, 0)),
            pl.BlockSpec((_MBLK, _H), lambda i: (i, 0)),
            pl.BlockSpec((_MBLK, _H), lambda i: (i, 0)),
            pl.BlockSpec((_MBLK, _H), lambda i: (i, 0)),
            pl.BlockSpec((_MBLK, _H), lambda i: (i, 0)),
            pl.BlockSpec((_H, _H), lambda i: (0, 0)),
            pl.BlockSpec((1, _H), lambda i: (0, 0)),
            pl.BlockSpec((_MBLK, _H), lambda i: (i, 0)),
            pl.BlockSpec((_MBLK, _H), lambda i: (i, 0)),
            pl.BlockSpec((_H, _OUT), lambda i: (0, 0)),
            pl.BlockSpec((_H, _OUT), lambda i: (0, 0)),
            pl.BlockSpec((_H, _OUT), lambda i: (0, 0)),
            pl.BlockSpec((1, _OUT), lambda i: (0, 0)),
        ],
        out_specs=pl.BlockSpec((_MBLK, _OUT), lambda i: (i, 0)),
        out_shape=jax.ShapeDtypeStruct((_NPAD, _OUT), jnp.float32),
    )(s0, s1, c0, c1, h, wr, bl, h1, h2, w1, w2, w3, b)


def kernel(x, edge_index, Wl0, bl0, Wr0, Wl1, bl1, Wr1, Wl2, bl2, Wr2,
           W_fc, b_fc):
    fill = _N + jnp.arange(_EPAD - _E, dtype=jnp.int32) % (_NPAD - _N)
    src2d = jnp.concatenate([edge_index[0], fill]).reshape(_EROWS, _CH)
    dst2d = jnp.concatenate([edge_index[1], fill]).reshape(_EROWS, _CH)

    xpad = jnp.pad(x, ((0, _NPAD - _N), (0, 0)))
    zeros_h = jnp.zeros((_ROWS_PT, _H), jnp.float32)
    ones_h = jnp.ones((_CH, _H), jnp.float32)

    cnt = _cnt_call()(dst2d, ones_h, zeros_h)

    p = _mm_p(xpad, Wl0)
    ssum = _seg_call()(p, src2d, dst2d, zeros_h)
    h1, p = _combine_proj(ssum[0], ssum[1], cnt[0], cnt[1], xpad, Wr0,
                          bl0.reshape(1, _H), Wl1)
    ssum = _seg_call()(p, src2d, dst2d, zeros_h)
    h2, p = _combine_proj(ssum[0], ssum[1], cnt[0], cnt[1], h1, Wr1,
                          bl1.reshape(1, _H), Wl2)
    ssum = _seg_call()(p, src2d, dst2d, zeros_h)
    out = _combine_fc(ssum[0], ssum[1], cnt[0], cnt[1], h2, Wr2,
                      bl2.reshape(1, _H), h1, h2, W_fc[0:_H],
                      W_fc[_H:2 * _H], W_fc[2 * _H:3 * _H],
                      b_fc.reshape(1, _OUT))
    return out[:_N]


# speedup vs baseline: 2.5895x; 1.1160x over previous
import functools

import jax
import jax.numpy as jnp
from jax import lax
from jax.experimental import pallas as pl
from jax.experimental.pallas import tpu as pltpu
from jax.experimental.pallas import tpu_sc as plsc

_N = 10000
_NPAD = 10240
_E = 320000
_F = 128
_H = 128
_OUT = 64
_NC = 2
_NS = 16
_CH = 80
_WPT = 128
_WPH = 64
_EROWS = _NC * _NS * _WPT
_EPAD = _EROWS * _CH
_ROWS_PT = _NPAD // _NS
_MBLK = 128


def _seg_body(p, src, dst, zeros, out,
              srcs, dsts, rows0, rows1, rows2, acc,
              sg0, sg1, sg2, ss0, ss1, ss2):
    c = lax.axis_index("c")
    s = lax.axis_index("s")
    t = c * _NS + s

    pltpu.sync_copy(zeros, acc.at[pl.ds(s * _ROWS_PT, _ROWS_PT)])
    plsc.subcore_barrier()

    rows = (rows0, rows1, rows2)
    sg = (sg0, sg1, sg2)
    ss = (ss0, ss1, ss2)

    def _gather(g, b):
        return pltpu.async_copy(p.at[srcs.at[g]], rows[b], sg[b])

    def _scatter(g, b):
        return pltpu.async_copy(rows[b], acc.at[dsts.at[g]], ss[b],
                                add=True)

    def _step(g, b, issue_ahead):
        pltpu.make_async_copy(p.at[srcs.at[g]], rows[b], sg[b]).wait()
        _scatter(g, b)
        pltpu.make_async_copy(rows[(b - 1) % 3],
                              acc.at[dsts.at[g - 1]], ss[(b - 1) % 3]).wait()
        if issue_ahead:
            _gather(g + 2, (b + 2) % 3)

    for ph in range(2):
        row0 = t * _WPT + ph * _WPH
        pltpu.sync_copy(src.at[pl.ds(row0, _WPH)], srcs)
        pltpu.sync_copy(dst.at[pl.ds(row0, _WPH)], dsts)

        _gather(0, 0)
        _gather(1, 1)
        pltpu.make_async_copy(p.at[srcs.at[0]], rows0, sg0).wait()
        _scatter(0, 0)
        _gather(2, 2)

        def _triple(i, _):
            g = 3 * i + 1
            _step(g, 1, True)
            _step(g + 1, 2, True)
            _step(g + 2, 0, True)
            return 0

        lax.fori_loop(0, (_WPH - 4) // 3, _triple, 0)

        _step(_WPH - 3, (_WPH - 3) % 3, True)
        _step(_WPH - 2, (_WPH - 2) % 3, False)
        _step(_WPH - 1, (_WPH - 1) % 3, False)
        b = (_WPH - 1) % 3
        pltpu.make_async_copy(rows[b], acc.at[dsts.at[_WPH - 1]],
                              ss[b]).wait()

    plsc.subcore_barrier()
    pltpu.sync_copy(acc.at[pl.ds(s * _ROWS_PT, _ROWS_PT)],
                    out.at[c, pl.ds(s * _ROWS_PT, _ROWS_PT)])


def _cnt_body(dst, ones, zeros, out, dsts, ones_v, acc, ss):
    c = lax.axis_index("c")
    s = lax.axis_index("s")
    t = c * _NS + s

    pltpu.sync_copy(ones, ones_v)
    pltpu.sync_copy(zeros, acc.at[pl.ds(s * _ROWS_PT, _ROWS_PT)])
    pltpu.sync_copy(dst.at[pl.ds(t * _WPT, _WPT)], dsts)
    plsc.subcore_barrier()

    def _grp(i, _):
        for j in range(8):
            pltpu.async_copy(ones_v, acc.at[dsts.at[8 * i + j]], ss,
                             add=True)
        for j in range(8):
            pltpu.make_async_copy(ones_v, acc.at[dsts.at[8 * i + j]],
                                  ss).wait()
        return 0

    lax.fori_loop(0, _WPT // 8, _grp, 0)
    plsc.subcore_barrier()
    pltpu.sync_copy(acc.at[pl.ds(s * _ROWS_PT, _ROWS_PT)],
                    out.at[c, pl.ds(s * _ROWS_PT, _ROWS_PT)])


@functools.cache
def _seg_call():
    mesh = plsc.VectorSubcoreMesh(core_axis_name="c", subcore_axis_name="s",
                                  num_cores=_NC, num_subcores=_NS)
    return pl.kernel(
        _seg_body,
        out_type=jax.ShapeDtypeStruct((_NC, _NPAD, _H), jnp.float32),
        mesh=mesh,
        scratch_types=[
            pltpu.VMEM((_WPH, _CH), jnp.int32),
            pltpu.VMEM((_WPH, _CH), jnp.int32),
            pltpu.VMEM((_CH, _H), jnp.float32),
            pltpu.VMEM((_CH, _H), jnp.float32),
            pltpu.VMEM((_CH, _H), jnp.float32),
            pltpu.VMEM_SHARED((_NPAD, _H), jnp.float32),
            pltpu.SemaphoreType.DMA,
            pltpu.SemaphoreType.DMA,
            pltpu.SemaphoreType.DMA,
            pltpu.SemaphoreType.DMA,
            pltpu.SemaphoreType.DMA,
            pltpu.SemaphoreType.DMA,
        ],
    )


@functools.cache
def _cnt_call():
    mesh = plsc.VectorSubcoreMesh(core_axis_name="c", subcore_axis_name="s",
                                  num_cores=_NC, num_subcores=_NS)
    return pl.kernel(
        _cnt_body,
        out_type=jax.ShapeDtypeStruct((_NC, _NPAD, _H), jnp.float32),
        mesh=mesh,
        scratch_types=[
            pltpu.VMEM((_WPT, _CH), jnp.int32),
            pltpu.VMEM((_CH, _H), jnp.float32),
            pltpu.VMEM_SHARED((_NPAD, _H), jnp.float32),
            pltpu.SemaphoreType.DMA,
        ],
    )


def _mm_p_kernel(h_ref, w_ref, o_ref):
    o_ref[...] = jnp.dot(h_ref[...], w_ref[...],
                         preferred_element_type=jnp.float32)


def _mm_p(h, wl):
    return pl.pallas_call(
        _mm_p_kernel,
        grid=(_NPAD // _MBLK,),
        in_specs=[
            pl.BlockSpec((_MBLK, _H), lambda i: (i, 0)),
            pl.BlockSpec((_H, _H), lambda i: (0, 0)),
        ],
        out_specs=pl.BlockSpec((_MBLK, _H), lambda i: (i, 0)),
        out_shape=jax.ShapeDtypeStruct((_NPAD, _H), jnp.float32),
    )(h, wl)


def _combine_proj_kernel(s0, s1, c0, c1, h, wr, bl, wl_n, h_ref, p_ref):
    cnt = c0[:, 0:1] + c1[:, 0:1]
    inv = 1.0 / jnp.maximum(cnt, 1.0)
    mean = (s0[...] + s1[...]) * inv
    mm = jnp.dot(h[...], wr[...], preferred_element_type=jnp.float32)
    hn = jnp.maximum(mean + mm + bl[...], 0.0)
    h_ref[...] = hn
    p_ref[...] = jnp.dot(hn, wl_n[...], preferred_element_type=jnp.float32)


def _combine_proj(s0, s1, c0, c1, h, wr, bl, wl_n):
    return pl.pallas_call(
        _combine_proj_kernel,
        grid=(_NPAD // _MBLK,),
        in_specs=[
            pl.BlockSpec((_MBLK, _H), lambda i: (i, 0)),
            pl.BlockSpec((_MBLK, _H), lambda i: (i, 0)),
            pl.BlockSpec((_MBLK, _H), lambda i: (i, 0)),
            pl.BlockSpec((_MBLK, _H), lambda i: (i, 0)),
            pl.BlockSpec((_MBLK, _H), lambda i: (i, 0)),
            pl.BlockSpec((_H, _H), lambda i: (0, 0)),
            pl.BlockSpec((1, _H), lambda i: (0, 0)),
            pl.BlockSpec((_H, _H), lambda i: (0, 0)),
        ],
        out_specs=[
            pl.BlockSpec((_MBLK, _H), lambda i: (i, 0)),
            pl.BlockSpec((_MBLK, _H), lambda i: (i, 0)),
        ],
        out_shape=[
            jax.ShapeDtypeStruct((_NPAD, _H), jnp.float32),
            jax.ShapeDtypeStruct((_NPAD, _H), jnp.float32),
        ],
    )(s0, s1, c0, c1, h, wr, bl, wl_n)


def _combine_fc_kernel(s0, s1, c0, c1, h, wr, bl, h1, h2,
                       w1, w2, w3, b, o_ref):
    cnt = c0[:, 0:1] + c1[:, 0:1]
    inv = 1.0 / jnp.maximum(cnt, 1.0)
    mean = (s0[...] + s1[...]) * inv
    mm = jnp.dot(h[...], wr[...], preferred_element_type=jnp.float32)
    h3 = jnp.maximum(mean + mm + bl[...], 0.0)
    acc = jnp.dot(h1[...], w1[...], preferred_element_type=jnp.float32)
    acc += jnp.dot(h2[...], w2[...], preferred_element_type=jnp.float32)
    acc += jnp.dot(h3, w3[...], preferred_element_type=jnp.float32)
    o_ref[...] = acc + b[...]


def _combine_fc(s0, s1, c0, c1, h, wr, bl, h1, h2, w1, w2, w3, b):
    return pl.pallas_call(
        _combine_fc_kernel,
        grid=(_NPAD // _MBLK,),
        in_specs=[
            pl.BlockSpec((_MBLK, _H), lambda i: (i, 0)),
            pl.BlockSpec((_MBLK, _H), lambda i: (i, 0)),
            pl.BlockSpec((_MBLK, _H), lambda i: (i, 0)),
            pl.BlockSpec((_MBLK, _H), lambda i: (i, 0)),
            pl.BlockSpec((_MBLK, _H), lambda i: (i, 0)),
            pl.BlockSpec((_H, _H), lambda i: (0, 0)),
            pl.BlockSpec((1, _H), lambda i: (0, 0)),
            pl.BlockSpec((_MBLK, _H), lambda i: (i, 0)),
            pl.BlockSpec((_MBLK, _H), lambda i: (i, 0)),
            pl.BlockSpec((_H, _OUT), lambda i: (0, 0)),
            pl.BlockSpec((_H, _OUT), lambda i: (0, 0)),
            pl.BlockSpec((_H, _OUT), lambda i: (0, 0)),
            pl.BlockSpec((1, _OUT), lambda i: (0, 0)),
        ],
        out_specs=pl.BlockSpec((_MBLK, _OUT), lambda i: (i, 0)),
        out_shape=jax.ShapeDtypeStruct((_NPAD, _OUT), jnp.float32),
    )(s0, s1, c0, c1, h, wr, bl, h1, h2, w1, w2, w3, b)


def kernel(x, edge_index, Wl0, bl0, Wr0, Wl1, bl1, Wr1, Wl2, bl2, Wr2,
           W_fc, b_fc):
    fill = _N + jnp.arange(_EPAD - _E, dtype=jnp.int32) % (_NPAD - _N)
    src2d = jnp.concatenate([edge_index[0], fill]).reshape(_EROWS, _CH)
    dst2d = jnp.concatenate([edge_index[1], fill]).reshape(_EROWS, _CH)

    xpad = jnp.pad(x, ((0, _NPAD - _N), (0, 0)))
    zeros_h = jnp.zeros((_ROWS_PT, _H), jnp.float32)
    ones_h = jnp.ones((_CH, _H), jnp.float32)

    cnt = _cnt_call()(dst2d, ones_h, zeros_h)

    p = _mm_p(xpad, Wl0)
    ssum = _seg_call()(p, src2d, dst2d, zeros_h)
    h1, p = _combine_proj(ssum[0], ssum[1], cnt[0], cnt[1], xpad, Wr0,
                          bl0.reshape(1, _H), Wl1)
    ssum = _seg_call()(p, src2d, dst2d, zeros_h)
    h2, p = _combine_proj(ssum[0], ssum[1], cnt[0], cnt[1], h1, Wr1,
                          bl1.reshape(1, _H), Wl2)
    ssum = _seg_call()(p, src2d, dst2d, zeros_h)
    out = _combine_fc(ssum[0], ssum[1], cnt[0], cnt[1], h2, Wr2,
                      bl2.reshape(1, _H), h1, h2, W_fc[0:_H],
                      W_fc[_H:2 * _H], W_fc[2 * _H:3 * _H],
                      b_fc.reshape(1, _OUT))
    return out[:_N]
